# baseline (speedup 1.0000x reference)
import functools

import jax
import jax.numpy as jnp
from jax import lax
from jax.experimental import pallas as pl
from jax.experimental.pallas import tpu as pltpu
from jax.experimental.pallas import tpu_sc as plsc

B, D, H, E, C = 4096, 1024, 1024, 8, 1024
NC, NS, L = 2, 16, 16
NW = NC * NS
CHUNK = B // NW
RG = CHUNK // L
TM = 256
TMS = 8
NTP = B // TM + E - 1
BP = NTP * TM
assert NTP <= 2 * L


def _mesh():
    return plsc.VectorSubcoreMesh(
        core_axis_name="c", subcore_axis_name="s", num_cores=NC, num_subcores=NS
    )


def _wid():
    return lax.axis_index("s") * NC + lax.axis_index("c")


def _logits_body(x_ref, wr_ref, br_ref, out_ref):
    out_ref[...] = (
        jnp.dot(x_ref[...], wr_ref[...], preferred_element_type=jnp.float32)
        + br_ref[...]
    )


def _feat_body(x_ref, wf_ref, bf_ref, out_ref):
    f = jnp.dot(x_ref[...], wf_ref[...], preferred_element_type=jnp.float32)
    out_ref[...] = jnp.maximum(f + bf_ref[...], 0.0)


def _ra_body(logits_hbm, choices_hbm, counts_hbm, log_v, ch_v, cnt_v):
    base = _wid() * CHUNK
    pltpu.sync_copy(logits_hbm.at[pl.ds(base, CHUNK)], log_v)
    lane = lax.iota(jnp.int32, L)
    ch_regs = []
    for j in range(RG):
        toks = jnp.full((L,), j * L, jnp.int32) + lane
        best = plsc.load_gather(log_v, [toks, jnp.zeros((L,), jnp.int32)])
        arg = jnp.zeros((L,), jnp.int32)
        for e in range(1, E):
            v = plsc.load_gather(log_v, [toks, jnp.full((L,), e, jnp.int32)])
            upd = v > best
            arg = jnp.where(upd, e, arg)
            best = jnp.where(upd, v, best)
        ch_regs.append(arg)
        ch_v[pl.ds(j * L, L)] = arg
    cnts = jnp.zeros((L,), jnp.int32)
    for j in range(RG):
        for e in range(E):
            c = plsc.all_reduce_population_count(ch_regs[j] == e)
            cnts = cnts + jnp.where(lane == e, c, 0)
    cnt_v[...] = cnts
    pltpu.sync_copy(ch_v, choices_hbm.at[pl.ds(base, CHUNK)])
    pltpu.sync_copy(cnt_v, counts_hbm.at[_wid()])


_SCAT = 32
_NB = CHUNK // _SCAT


def _rb_body(choices_hbm, counts_hbm, feat_hbm,
             inv_hbm, sf_hbm, wk_hbm,
             ch_v, cnts_v, inv_v, wk_v, idx0, idx1, idx2, idx3,
             fbuf, sem):
    wid = _wid()
    base = wid * CHUNK
    pltpu.sync_copy(choices_hbm.at[pl.ds(base, CHUNK)], ch_v)
    pltpu.sync_copy(counts_hbm, cnts_v)
    lane = lax.iota(jnp.int32, L)
    totals = jnp.zeros((L,), jnp.int32)
    prefix = jnp.zeros((L,), jnp.int32)
    for w in range(NW):
        row = cnts_v[w, :]
        totals = totals + row
        prefix = prefix + jnp.where(w < wid, row, 0)
    pcnt = lax.shift_left(
        lax.shift_right_logical(totals + (TM - 1), TMS), TMS)
    ebase = plsc.cumsum(pcnt) - pcnt

    running = ebase + prefix
    idx_bufs = (idx0, idx1, idx2, idx3)
    for j in range(RG):
        v = ch_v[pl.ds(j * L, L)]
        dest = jnp.zeros((L,), jnp.int32)
        for e in range(E):
            m = v == e
            ones = m.astype(jnp.int32)
            rank = plsc.cumsum(ones) - 1
            base_e = jnp.sum(jnp.where(lane == e, running, 0))
            dest = jnp.where(m, base_e + rank, dest)
            c = plsc.all_reduce_population_count(m)
            running = running + jnp.where(lane == e, c, 0)
        inv_v[pl.ds(j * L, L)] = dest
        idx_bufs[j // 2][pl.ds((j % 2) * L, L)] = dest
    pltpu.sync_copy(inv_v, inv_hbm.at[pl.ds(base, CHUNK)])
    for k in range(_NB):
        pltpu.sync_copy(feat_hbm.at[pl.ds(base + k * _SCAT, _SCAT)], fbuf)
        pltpu.async_copy(fbuf, sf_hbm.at[idx_bufs[k]], sem).wait()

    @pl.when(wid == 0)
    def _():
        for half in range(2):
            tstart = (lane + half * L) * TM
            emap = jnp.full((L,), -1, jnp.int32)
            for e in range(E + 1):
                off_e = jnp.sum(jnp.where(lane == e, ebase, 0))
                emap = emap + (off_e <= tstart).astype(jnp.int32)
            wk_v[pl.ds(half * L, L)] = jnp.minimum(emap, E - 1)
        pltpu.sync_copy(wk_v, wk_hbm)


def _g_body(wk_r, fs_ref, lw_ref, lb_ref, out_ref):
    out_ref[...] = (
        jnp.dot(fs_ref[...], lw_ref[0], preferred_element_type=jnp.float32)
        + lb_ref[0]
    )


def _u_body(so_hbm, inv_hbm, out_hbm, idx_v, buf, sem):
    base = _wid() * CHUNK
    for k in range(_NB):
        pltpu.sync_copy(inv_hbm.at[pl.ds(base + k * _SCAT, _SCAT)], idx_v)
        pltpu.async_copy(so_hbm.at[idx_v], buf, sem).wait()
        pltpu.sync_copy(buf, out_hbm.at[pl.ds(base + k * _SCAT, _SCAT)])


def kernel(inputs, W_feat, b_feat, W_route, b_route, leaf_W, leaf_b):
    logits = pl.pallas_call(
        _logits_body,
        grid=(4,),
        in_specs=[
            pl.BlockSpec((B // 4, D), lambda i: (i, 0)),
            pl.BlockSpec((D, E), lambda i: (0, 0)),
            pl.BlockSpec((1, E), lambda i: (0, 0)),
        ],
        out_specs=pl.BlockSpec((B // 4, E), lambda i: (i, 0)),
        out_shape=jax.ShapeDtypeStruct((B, E), jnp.float32),
    )(inputs, W_route, b_route.reshape(1, E))

    feat = pl.pallas_call(
        _feat_body,
        grid=(8,),
        in_specs=[
            pl.BlockSpec((B // 8, D), lambda i: (i, 0)),
            pl.BlockSpec((D, H), lambda i: (0, 0)),
            pl.BlockSpec((1, H), lambda i: (0, 0)),
        ],
        out_specs=pl.BlockSpec((B // 8, H), lambda i: (i, 0)),
        out_shape=jax.ShapeDtypeStruct((B, H), jnp.float32),
    )(inputs, W_feat, b_feat.reshape(1, H))

    choices, counts = pl.kernel(
        _ra_body,
        out_type=[
            jax.ShapeDtypeStruct((B,), jnp.int32),
            jax.ShapeDtypeStruct((NW, L), jnp.int32),
        ],
        mesh=_mesh(),
        compiler_params=pltpu.CompilerParams(needs_layout_passes=False),
        scratch_types=[
            pltpu.VMEM((CHUNK, E), jnp.float32),
            pltpu.VMEM((CHUNK,), jnp.int32),
            pltpu.VMEM((L,), jnp.int32),
        ],
    )(logits)

    inv, sorted_feat, wk = pl.kernel(
        _rb_body,
        out_type=[
            jax.ShapeDtypeStruct((B,), jnp.int32),
            jax.ShapeDtypeStruct((BP, H), jnp.float32),
            jax.ShapeDtypeStruct((2 * L,), jnp.int32),
        ],
        mesh=_mesh(),
        compiler_params=pltpu.CompilerParams(needs_layout_passes=False),
        scratch_types=[
            pltpu.VMEM((CHUNK,), jnp.int32),
            pltpu.VMEM((NW, L), jnp.int32),
            pltpu.VMEM((CHUNK,), jnp.int32),
            pltpu.VMEM((2 * L,), jnp.int32),
            pltpu.VMEM((_SCAT,), jnp.int32),
            pltpu.VMEM((_SCAT,), jnp.int32),
            pltpu.VMEM((_SCAT,), jnp.int32),
            pltpu.VMEM((_SCAT,), jnp.int32),
            pltpu.VMEM((_SCAT, H), jnp.float32),
            pltpu.SemaphoreType.DMA,
        ],
    )(choices, counts, feat)

    sorted_out = pl.pallas_call(
        _g_body,
        grid_spec=pltpu.PrefetchScalarGridSpec(
            num_scalar_prefetch=1,
            grid=(NTP,),
            in_specs=[
                pl.BlockSpec((TM, H), lambda w, wk: (w, 0)),
                pl.BlockSpec((1, H, C), lambda w, wk: (wk[w], 0, 0)),
                pl.BlockSpec((1, 1, C), lambda w, wk: (wk[w], 0, 0)),
            ],
            out_specs=pl.BlockSpec((TM, C), lambda w, wk: (w, 0)),
        ),
        out_shape=jax.ShapeDtypeStruct((BP, C), jnp.float32),
    )(wk, sorted_feat, leaf_W, leaf_b.reshape(E, 1, C))

    predictions = pl.kernel(
        _u_body,
        out_type=jax.ShapeDtypeStruct((B, C), jnp.float32),
        mesh=_mesh(),
        compiler_params=pltpu.CompilerParams(needs_layout_passes=False),
        scratch_types=[
            pltpu.VMEM((_SCAT,), jnp.int32),
            pltpu.VMEM((_SCAT, C), jnp.float32),
            pltpu.SemaphoreType.DMA,
        ],
    )(sorted_out, inv)
    return predictions
